# baseline (speedup 1.0000x reference)
import functools

import jax
import jax.numpy as jnp
from jax import lax
from jax.experimental import pallas as pl
from jax.experimental.pallas import tpu as pltpu
from jax.experimental.pallas import tpu_sc as plsc

_DIM = 1024
_HEADS = 8
_DIM_HEAD = 64
_G = 2
_NQ = 1024
_NKV = 1024
_N_ITERS = 20
_FETCH_K_RATIO = 9.0 / 8.0
_EPS = 0.03
_EPS_INIT = 4.0
_EPS_DECAY = 0.7
_DI = _HEADS * _DIM_HEAD

_INTERPRET = False


_NW = 32
_CHUNK = 64


def _sc_gather_rows(table, idx):
    nrows = idx.shape[0]
    b_per_w = nrows // _NW
    nch = b_per_w // _CHUNK
    mesh = plsc.VectorSubcoreMesh(core_axis_name="c", subcore_axis_name="s")

    @functools.partial(
        pl.kernel, mesh=mesh,
        out_type=jax.ShapeDtypeStruct((nrows, _DIM), jnp.float32),
        scratch_types=[
            pltpu.VMEM((b_per_w,), jnp.int32),
            pltpu.VMEM((_CHUNK, _DIM), jnp.float32),
            pltpu.SemaphoreType.DMA,
        ],
    )
    def k(table_hbm, idx_hbm, out_hbm, idx_v, rows_v, gsem):
        wid = lax.axis_index("s") * 2 + lax.axis_index("c")
        base = wid * b_per_w
        pltpu.sync_copy(idx_hbm.at[pl.ds(base, b_per_w)], idx_v)
        for c in range(nch):
            pltpu.async_copy(
                table_hbm.at[idx_v.at[pl.ds(c * _CHUNK, _CHUNK)]],
                rows_v, gsem).wait()
            pltpu.sync_copy(rows_v,
                            out_hbm.at[pl.ds(base + c * _CHUNK, _CHUNK)])

    return k(table, idx)


_RCHUNK = 16


def _sc_recombine(tab, idx0, idx1):
    nrows = idx0.shape[0]
    b_per_w = nrows // _NW
    nch = b_per_w // _RCHUNK
    mesh = plsc.VectorSubcoreMesh(core_axis_name="c", subcore_axis_name="s")

    @functools.partial(
        pl.kernel, mesh=mesh,
        out_type=jax.ShapeDtypeStruct((nrows, _DIM), jnp.float32),
        scratch_types=[
            pltpu.VMEM((b_per_w,), jnp.int32),
            pltpu.VMEM((b_per_w,), jnp.int32),
            pltpu.VMEM((_RCHUNK, _DIM), jnp.float32),
            pltpu.VMEM((_RCHUNK, _DIM), jnp.float32),
            pltpu.VMEM((_RCHUNK, _DIM), jnp.float32),
            pltpu.VMEM((_RCHUNK, _DIM), jnp.float32),
            pltpu.SemaphoreType.DMA,
            pltpu.SemaphoreType.DMA,
            pltpu.SemaphoreType.DMA,
            pltpu.SemaphoreType.DMA,
        ],
    )
    def k(tab_hbm, i0_hbm, i1_hbm, out_hbm, i0_v, i1_v,
          r0a, r0b, r1a, r1b, sem0a, sem0b, sem1a, sem1b):
        wid = lax.axis_index("s") * 2 + lax.axis_index("c")
        base = wid * b_per_w
        pltpu.sync_copy(i0_hbm.at[pl.ds(base, b_per_w)], i0_v)
        pltpu.sync_copy(i1_hbm.at[pl.ds(base, b_per_w)], i1_v)
        r0 = (r0a, r0b)
        r1 = (r1a, r1b)
        s0 = (sem0a, sem0b)
        s1 = (sem1a, sem1b)

        def issue(c):
            s = c % 2
            cp0 = pltpu.async_copy(
                tab_hbm.at[i0_v.at[pl.ds(c * _RCHUNK, _RCHUNK)]],
                r0[s], s0[s])
            cp1 = pltpu.async_copy(
                tab_hbm.at[i1_v.at[pl.ds(c * _RCHUNK, _RCHUNK)]],
                r1[s], s1[s])
            return cp0, cp1

        cps = issue(0)
        for c in range(nch):
            s = c % 2
            nxt = issue(c + 1) if c + 1 < nch else None
            cps[0].wait()
            cps[1].wait()

            def row_body(row, _, _s=s):
                for j in range(_DIM // 16):
                    sl = pl.ds(j * 16, 16)
                    r0[_s][row, sl] = r0[_s][row, sl] + r1[_s][row, sl]
                return 0

            lax.fori_loop(0, _RCHUNK, row_body, 0)
            pltpu.sync_copy(r0[s],
                            out_hbm.at[pl.ds(base + c * _RCHUNK, _RCHUNK)])
            cps = nxt

    return k(tab, idx0, idx1)


def _coor_descent(s, k):
    logk = jnp.log(jnp.asarray(float(k), dtype=s.dtype))
    a = jnp.zeros_like(s[..., :1])
    bneg = -s
    current_eps = max(_EPS_INIT, _EPS)
    for _ in range(_N_ITERS):
        sb = (s + bneg) / current_eps
        a = current_eps * (logk - jax.nn.logsumexp(sb, axis=-1, keepdims=True))
        bneg = -jax.nn.relu(s + a)
        current_eps = max(current_eps * _EPS_DECAY, _EPS)
    return jnp.exp((s + a + bneg) / current_eps)


def _route_idx(x, routing_token, num_tokens):
    bsz, n, _ = x.shape
    s = jnp.einsum('bnd,gd->bgn', x, routing_token)
    k = min(int(num_tokens * _FETCH_K_RATIO), n)
    scores = _coor_descent(s, k)
    g = scores.shape[1]
    sat = scores >= 1.0
    cs = jnp.cumsum(sat.astype(jnp.int32), axis=-1)
    enough = jnp.min(cs[:, :, -1]) >= num_tokens

    def fast(_):
        dest = jnp.where(sat & (cs <= num_tokens), cs - 1, num_tokens)
        iota = jnp.broadcast_to(jnp.arange(n, dtype=jnp.int32),
                                (bsz, g, n))
        buf = jnp.zeros((bsz, g, num_tokens + 1), jnp.int32)
        buf = buf.at[jnp.arange(bsz)[:, None, None],
                     jnp.arange(g)[None, :, None], dest].set(iota)
        return buf[..., :num_tokens]

    def slow(_):
        return jax.lax.top_k(scores, num_tokens)[1]

    return lax.cond(enough, fast, slow, 0)


def _dotT(a, b):
    return lax.dot_general(a, b, (((1,), (1,)), ((), ())),
                           precision=lax.Precision.DEFAULT,
                           preferred_element_type=jnp.float32)


def _rms(t, gamma):
    ss = jnp.sum(t * t, axis=1, keepdims=True)
    normed = t / jnp.maximum(jnp.sqrt(ss), 1e-12)
    return normed * (float(_DIM) ** 0.5) * gamma


def _proj_body(qtok_ref, kvtok_ref, gx_ref, gc_ref, wq_ref, wkv_ref,
               q_ref, k_ref, v_ref):
    qn = _rms(qtok_ref[0, 0], gx_ref[...].reshape(1, _DIM))
    q_ref[0, 0] = _dotT(qn, wq_ref[0]).astype(jnp.bfloat16)
    tn = _rms(kvtok_ref[0, 0], gc_ref[...].reshape(1, _DIM))
    k_ref[0, 0] = _dotT(tn, wkv_ref[0, :_DI]).astype(jnp.bfloat16)
    v_ref[0, 0] = _dotT(tn, wkv_ref[0, _DI:]).astype(jnp.bfloat16)


_QB = 512


def _attn_body(q_ref, k_ref, v_ref, nk_ref, nv_ref, wo_ref, inv_ref,
               out_ref, o_scratch):
    qs = q_ref[0, 0] * jnp.bfloat16(_DIM_HEAD ** -0.5)
    kk = k_ref[0, 0]
    vv = v_ref[0, 0]
    nk = nk_ref[0]
    nv = nv_ref[0]
    for h in range(_HEADS):
        sl = slice(h * _DIM_HEAD, (h + 1) * _DIM_HEAD)
        qh = qs[:, sl]
        sim = _dotT(qh, kk[:, sl])
        simn = jnp.sum(qh.astype(jnp.float32) * nk[h][None, :],
                       axis=1, keepdims=True)
        m = jnp.maximum(jnp.max(sim, axis=1, keepdims=True), simn)
        p = jnp.exp(sim - m)
        pn = jnp.exp(simn - m)
        denom = jnp.sum(p, axis=1, keepdims=True) + pn
        ov = lax.dot_general(p.astype(jnp.bfloat16), vv[:, sl],
                             (((1,), (0,)), ((), ())),
                             precision=lax.Precision.DEFAULT,
                             preferred_element_type=jnp.float32)
        o_scratch[:, sl] = ((ov + pn * nv[h:h + 1]) / denom
                            ).astype(jnp.bfloat16)
    out = _dotT(o_scratch[...], wo_ref[0])
    out_ref[0, 0] = out * inv_ref[...].reshape(_QB, 1)


def _grouped_attention(q_tok, kv_tok, gamma_x, gamma_ctx, nk, nv,
                       Wq, Wkv, Wo, inv_cnt):
    b = q_tok.shape[0]
    q, kk, vv = pl.pallas_call(
        _proj_body,
        grid=(b, _G),
        in_specs=[
            pl.BlockSpec((1, 1, _NQ, _DIM), lambda i, j: (i, j, 0, 0)),
            pl.BlockSpec((1, 1, _NKV, _DIM), lambda i, j: (i, j, 0, 0)),
            pl.BlockSpec((1, 1, _DIM), lambda i, j: (j, 0, 0)),
            pl.BlockSpec((1, 1, _DIM), lambda i, j: (j, 0, 0)),
            pl.BlockSpec((1, _DI, _DIM), lambda i, j: (j, 0, 0)),
            pl.BlockSpec((1, 2 * _DI, _DIM), lambda i, j: (j, 0, 0)),
        ],
        out_specs=[
            pl.BlockSpec((1, 1, _NQ, _DI), lambda i, j: (i, j, 0, 0)),
            pl.BlockSpec((1, 1, _NKV, _DI), lambda i, j: (i, j, 0, 0)),
            pl.BlockSpec((1, 1, _NKV, _DI), lambda i, j: (i, j, 0, 0)),
        ],
        out_shape=[
            jax.ShapeDtypeStruct((b, _G, _NQ, _DI), jnp.bfloat16),
            jax.ShapeDtypeStruct((b, _G, _NKV, _DI), jnp.bfloat16),
            jax.ShapeDtypeStruct((b, _G, _NKV, _DI), jnp.bfloat16),
        ],
        interpret=_INTERPRET,
    )(q_tok, kv_tok, gamma_x, gamma_ctx, Wq, Wkv)
    nqb = _NQ // _QB
    return pl.pallas_call(
        _attn_body,
        grid=(b, _G, nqb),
        in_specs=[
            pl.BlockSpec((1, 1, _QB, _DI), lambda i, j, q_: (i, j, q_, 0)),
            pl.BlockSpec((1, 1, _NKV, _DI), lambda i, j, q_: (i, j, 0, 0)),
            pl.BlockSpec((1, 1, _NKV, _DI), lambda i, j, q_: (i, j, 0, 0)),
            pl.BlockSpec((1, _HEADS, _DIM_HEAD), lambda i, j, q_: (j, 0, 0)),
            pl.BlockSpec((1, _HEADS, _DIM_HEAD), lambda i, j, q_: (j, 0, 0)),
            pl.BlockSpec((1, _DIM, _DI), lambda i, j, q_: (j, 0, 0)),
            pl.BlockSpec((1, 1, 1, _QB), lambda i, j, q_: (i, j, 0, q_)),
        ],
        out_specs=pl.BlockSpec((1, 1, _QB, _DIM),
                               lambda i, j, q_: (i, j, q_, 0)),
        out_shape=jax.ShapeDtypeStruct((b, _G, _NQ, _DIM), jnp.float32),
        scratch_shapes=[pltpu.VMEM((_QB, _DI), jnp.bfloat16)],
        interpret=_INTERPRET,
    )(q, kk, vv, nk, nv, Wo.astype(jnp.bfloat16), inv_cnt)


def kernel(x, rt_q, rt_kv, gamma_x, gamma_ctx, null_kv, Wq, Wkv, Wo,
           null_routed_token):
    b, n, d = x.shape
    q_idx = _route_idx(x, rt_q, _NQ)
    kv_idx = _route_idx(x, rt_kv, _NKV)

    brows = jnp.arange(b)[:, None]

    flat_q = (jnp.arange(b, dtype=jnp.int32)[:, None, None] * n
              + q_idx).reshape(-1)
    flat_kv = (jnp.arange(b, dtype=jnp.int32)[:, None, None] * n
               + kv_idx).reshape(-1)
    rows = _sc_gather_rows(x.reshape(b * n, d),
                           jnp.concatenate([flat_q, flat_kv]))
    q_tok = rows[:b * _G * _NQ].reshape(b, _G, _NQ, d)
    kv_tok = rows[b * _G * _NQ:].reshape(b, _G, _NKV, d)

    qi_flat = q_idx.reshape(b, _G * _NQ)
    cnt = jnp.zeros((b, n), jnp.int32).at[brows, qi_flat].add(1)
    c_slot = jnp.take_along_axis(jnp.broadcast_to(cnt[:, None, :], (b, _G, n)),
                                 q_idx, axis=2)
    inv_cnt = (1.0 / c_slot.astype(jnp.float32)).reshape(b, _G, 1, _NQ)

    nk = null_kv[0, :, :, 0, :]
    nv = null_kv[1, :, :, 0, :]
    gx = gamma_x[:, None, :, 0]
    gc = gamma_ctx[:, None, :, 0]

    ao = _grouped_attention(q_tok, kv_tok, gx, gc, nk, nv, Wq, Wkv, Wo,
                            inv_cnt)

    slots = jnp.broadcast_to(
        jnp.arange(_NQ, dtype=jnp.int32)[None, None, :], q_idx.shape)
    pos = jnp.full((b, _G, n), -1, jnp.int32)
    pos = pos.at[jnp.arange(b)[:, None, None],
                 jnp.arange(_G)[None, :, None], q_idx].set(slots)
    pos0, pos1 = pos[:, 0], pos[:, 1]
    hit0, hit1 = pos0 >= 0, pos1 >= 0
    arena = 256
    zero_base = b * _G * _NQ
    null_base = zero_base + arena
    spread = (jnp.arange(n, dtype=jnp.int32) % arena)[None, :]
    base = brows * (_G * _NQ)
    idx0 = jnp.where(hit0, base + pos0,
                     jnp.where(hit1, zero_base + spread,
                               null_base + spread)).astype(jnp.int32)
    idx1 = jnp.where(hit1, base + _NQ + pos1,
                     zero_base + spread).astype(jnp.int32)

    tab = jnp.concatenate([
        ao.reshape(b * _G * _NQ, d),
        jnp.zeros((arena, d), jnp.float32),
        jnp.broadcast_to(null_routed_token.reshape(1, d),
                         (arena, d)).astype(jnp.float32),
    ], axis=0)
    out = _sc_recombine(tab, idx0.reshape(-1), idx1.reshape(-1))
    return out.reshape(b, n, d)


# speedup vs baseline: 1.2822x; 1.2822x over previous
import functools

import jax
import jax.numpy as jnp
from jax import lax
from jax.experimental import pallas as pl
from jax.experimental.pallas import tpu as pltpu
from jax.experimental.pallas import tpu_sc as plsc

_DIM = 1024
_HEADS = 8
_DIM_HEAD = 64
_G = 2
_NQ = 1024
_NKV = 1024
_N_ITERS = 20
_FETCH_K_RATIO = 9.0 / 8.0
_EPS = 0.03
_EPS_INIT = 4.0
_EPS_DECAY = 0.7
_DI = _HEADS * _DIM_HEAD

_INTERPRET = False


_NW = 32
_CHUNK = 64


def _sc_gather_rows(table, idx):
    nrows = idx.shape[0]
    b_per_w = nrows // _NW
    nch = b_per_w // _CHUNK
    mesh = plsc.VectorSubcoreMesh(core_axis_name="c", subcore_axis_name="s")

    @functools.partial(
        pl.kernel, mesh=mesh,
        out_type=jax.ShapeDtypeStruct((nrows, _DIM), jnp.float32),
        scratch_types=[
            pltpu.VMEM((b_per_w,), jnp.int32),
            pltpu.VMEM((_CHUNK, _DIM), jnp.float32),
            pltpu.SemaphoreType.DMA,
        ],
    )
    def k(table_hbm, idx_hbm, out_hbm, idx_v, rows_v, gsem):
        wid = lax.axis_index("s") * 2 + lax.axis_index("c")
        base = wid * b_per_w
        pltpu.sync_copy(idx_hbm.at[pl.ds(base, b_per_w)], idx_v)
        for c in range(nch):
            pltpu.async_copy(
                table_hbm.at[idx_v.at[pl.ds(c * _CHUNK, _CHUNK)]],
                rows_v, gsem).wait()
            pltpu.sync_copy(rows_v,
                            out_hbm.at[pl.ds(base + c * _CHUNK, _CHUNK)])

    return k(table, idx)


_RCHUNK = 16


def _sc_recombine(tab, idx0, idx1):
    nrows = idx0.shape[0]
    b_per_w = nrows // _NW
    nch = b_per_w // _RCHUNK
    mesh = plsc.VectorSubcoreMesh(core_axis_name="c", subcore_axis_name="s")

    @functools.partial(
        pl.kernel, mesh=mesh,
        out_type=jax.ShapeDtypeStruct((nrows, _DIM), jnp.float32),
        scratch_types=[
            pltpu.VMEM((b_per_w,), jnp.int32),
            pltpu.VMEM((b_per_w,), jnp.int32),
            pltpu.VMEM((_RCHUNK, _DIM), jnp.float32),
            pltpu.VMEM((_RCHUNK, _DIM), jnp.float32),
            pltpu.VMEM((_RCHUNK, _DIM), jnp.float32),
            pltpu.VMEM((_RCHUNK, _DIM), jnp.float32),
            pltpu.SemaphoreType.DMA,
            pltpu.SemaphoreType.DMA,
            pltpu.SemaphoreType.DMA,
            pltpu.SemaphoreType.DMA,
            pltpu.SemaphoreType.DMA,
            pltpu.SemaphoreType.DMA,
        ],
    )
    def k(tab_hbm, i0_hbm, i1_hbm, out_hbm, i0_v, i1_v,
          r0a, r0b, r1a, r1b, sem0a, sem0b, sem1a, sem1b, wsa, wsb):
        wid = lax.axis_index("s") * 2 + lax.axis_index("c")
        base = wid * b_per_w
        pltpu.sync_copy(i0_hbm.at[pl.ds(base, b_per_w)], i0_v)
        pltpu.sync_copy(i1_hbm.at[pl.ds(base, b_per_w)], i1_v)
        r0 = (r0a, r0b)
        r1 = (r1a, r1b)
        s0 = (sem0a, sem0b)
        s1 = (sem1a, sem1b)
        ws = (wsa, wsb)

        def issue(c):
            s = c % 2
            cp0 = pltpu.async_copy(
                tab_hbm.at[i0_v.at[pl.ds(c * _RCHUNK, _RCHUNK)]],
                r0[s], s0[s])
            cp1 = pltpu.async_copy(
                tab_hbm.at[i1_v.at[pl.ds(c * _RCHUNK, _RCHUNK)]],
                r1[s], s1[s])
            return cp0, cp1

        cps = issue(0)
        wcps = [None, None]
        for c in range(nch):
            s = c % 2
            if c + 1 < nch:
                if wcps[1 - s] is not None:
                    wcps[1 - s].wait()
                    wcps[1 - s] = None
                nxt = issue(c + 1)
            else:
                nxt = None
            cps[0].wait()
            cps[1].wait()

            def row_body(row, _, _s=s):
                for j in range(_DIM // 16):
                    sl = pl.ds(j * 16, 16)
                    r0[_s][row, sl] = r0[_s][row, sl] + r1[_s][row, sl]
                return 0

            lax.fori_loop(0, _RCHUNK, row_body, 0)
            wcps[s] = pltpu.async_copy(
                r0[s], out_hbm.at[pl.ds(base + c * _RCHUNK, _RCHUNK)], ws[s])
            cps = nxt
        for w in wcps:
            if w is not None:
                w.wait()

    return k(tab, idx0, idx1)


def _coor_descent(s, k):
    logk = jnp.log(jnp.asarray(float(k), dtype=s.dtype))
    a = jnp.zeros_like(s[..., :1])
    bneg = -s
    current_eps = max(_EPS_INIT, _EPS)
    for _ in range(_N_ITERS):
        sb = (s + bneg) / current_eps
        a = current_eps * (logk - jax.nn.logsumexp(sb, axis=-1, keepdims=True))
        bneg = -jax.nn.relu(s + a)
        current_eps = max(current_eps * _EPS_DECAY, _EPS)
    return jnp.exp((s + a + bneg) / current_eps)


def _route_idx(x, routing_token, num_tokens):
    bsz, n, _ = x.shape
    s = jnp.einsum('bnd,gd->bgn', x, routing_token)
    k = min(int(num_tokens * _FETCH_K_RATIO), n)
    scores = _coor_descent(s, k)
    _, sel_idx = jax.lax.top_k(scores, num_tokens)
    return sel_idx


def _dotT(a, b):
    return lax.dot_general(a, b, (((1,), (1,)), ((), ())),
                           precision=lax.Precision.DEFAULT,
                           preferred_element_type=jnp.float32)


def _rms(t, gamma):
    ss = jnp.sum(t * t, axis=1, keepdims=True)
    normed = t / jnp.maximum(jnp.sqrt(ss), 1e-12)
    return normed * (float(_DIM) ** 0.5) * gamma


def _proj_body(qtok_ref, kvtok_ref, gx_ref, gc_ref, wq_ref, wkv_ref,
               q_ref, k_ref, v_ref):
    qn = _rms(qtok_ref[0, 0], gx_ref[...].reshape(1, _DIM))
    q_ref[0, 0] = _dotT(qn, wq_ref[0]).astype(jnp.bfloat16)
    tn = _rms(kvtok_ref[0, 0], gc_ref[...].reshape(1, _DIM))
    k_ref[0, 0] = _dotT(tn, wkv_ref[0, :_DI]).astype(jnp.bfloat16)
    v_ref[0, 0] = _dotT(tn, wkv_ref[0, _DI:]).astype(jnp.bfloat16)


_QB = 512


def _attn_body(q_ref, k_ref, v_ref, nk_ref, nv_ref, wo_ref, inv_ref,
               out_ref, o_scratch):
    qs = q_ref[0, 0] * jnp.bfloat16(_DIM_HEAD ** -0.5)
    kk = k_ref[0, 0]
    vv = v_ref[0, 0]
    nk = nk_ref[0]
    nv = nv_ref[0]
    for h in range(_HEADS):
        sl = slice(h * _DIM_HEAD, (h + 1) * _DIM_HEAD)
        qh = qs[:, sl]
        sim = _dotT(qh, kk[:, sl])
        simn = jnp.sum(qh.astype(jnp.float32) * nk[h][None, :],
                       axis=1, keepdims=True)
        m = jnp.maximum(jnp.max(sim, axis=1, keepdims=True), simn)
        p = jnp.exp(sim - m)
        pn = jnp.exp(simn - m)
        denom = jnp.sum(p, axis=1, keepdims=True) + pn
        ov = lax.dot_general(p.astype(jnp.bfloat16), vv[:, sl],
                             (((1,), (0,)), ((), ())),
                             precision=lax.Precision.DEFAULT,
                             preferred_element_type=jnp.float32)
        o_scratch[:, sl] = ((ov + pn * nv[h:h + 1]) / denom
                            ).astype(jnp.bfloat16)
    out = _dotT(o_scratch[...], wo_ref[0])
    out_ref[0, 0] = out * inv_ref[...].reshape(_QB, 1)


def _grouped_attention(q_tok, kv_tok, gamma_x, gamma_ctx, nk, nv,
                       Wq, Wkv, Wo, inv_cnt):
    b = q_tok.shape[0]
    q, kk, vv = pl.pallas_call(
        _proj_body,
        grid=(b, _G),
        in_specs=[
            pl.BlockSpec((1, 1, _NQ, _DIM), lambda i, j: (i, j, 0, 0)),
            pl.BlockSpec((1, 1, _NKV, _DIM), lambda i, j: (i, j, 0, 0)),
            pl.BlockSpec((1, 1, _DIM), lambda i, j: (j, 0, 0)),
            pl.BlockSpec((1, 1, _DIM), lambda i, j: (j, 0, 0)),
            pl.BlockSpec((1, _DI, _DIM), lambda i, j: (j, 0, 0)),
            pl.BlockSpec((1, 2 * _DI, _DIM), lambda i, j: (j, 0, 0)),
        ],
        out_specs=[
            pl.BlockSpec((1, 1, _NQ, _DI), lambda i, j: (i, j, 0, 0)),
            pl.BlockSpec((1, 1, _NKV, _DI), lambda i, j: (i, j, 0, 0)),
            pl.BlockSpec((1, 1, _NKV, _DI), lambda i, j: (i, j, 0, 0)),
        ],
        out_shape=[
            jax.ShapeDtypeStruct((b, _G, _NQ, _DI), jnp.bfloat16),
            jax.ShapeDtypeStruct((b, _G, _NKV, _DI), jnp.bfloat16),
            jax.ShapeDtypeStruct((b, _G, _NKV, _DI), jnp.bfloat16),
        ],
        interpret=_INTERPRET,
    )(q_tok, kv_tok, gamma_x, gamma_ctx, Wq, Wkv)
    nqb = _NQ // _QB
    return pl.pallas_call(
        _attn_body,
        grid=(b, _G, nqb),
        in_specs=[
            pl.BlockSpec((1, 1, _QB, _DI), lambda i, j, q_: (i, j, q_, 0)),
            pl.BlockSpec((1, 1, _NKV, _DI), lambda i, j, q_: (i, j, 0, 0)),
            pl.BlockSpec((1, 1, _NKV, _DI), lambda i, j, q_: (i, j, 0, 0)),
            pl.BlockSpec((1, _HEADS, _DIM_HEAD), lambda i, j, q_: (j, 0, 0)),
            pl.BlockSpec((1, _HEADS, _DIM_HEAD), lambda i, j, q_: (j, 0, 0)),
            pl.BlockSpec((1, _DIM, _DI), lambda i, j, q_: (j, 0, 0)),
            pl.BlockSpec((1, 1, 1, _QB), lambda i, j, q_: (i, j, 0, q_)),
        ],
        out_specs=pl.BlockSpec((1, 1, _QB, _DIM),
                               lambda i, j, q_: (i, j, q_, 0)),
        out_shape=jax.ShapeDtypeStruct((b, _G, _NQ, _DIM), jnp.float32),
        scratch_shapes=[pltpu.VMEM((_QB, _DI), jnp.bfloat16)],
        interpret=_INTERPRET,
    )(q, kk, vv, nk, nv, Wo.astype(jnp.bfloat16), inv_cnt)


def kernel(x, rt_q, rt_kv, gamma_x, gamma_ctx, null_kv, Wq, Wkv, Wo,
           null_routed_token):
    b, n, d = x.shape
    q_idx = _route_idx(x, rt_q, _NQ)
    kv_idx = _route_idx(x, rt_kv, _NKV)

    brows = jnp.arange(b)[:, None]

    flat_q = (jnp.arange(b, dtype=jnp.int32)[:, None, None] * n
              + q_idx).reshape(-1)
    flat_kv = (jnp.arange(b, dtype=jnp.int32)[:, None, None] * n
               + kv_idx).reshape(-1)
    rows = _sc_gather_rows(x.reshape(b * n, d),
                           jnp.concatenate([flat_q, flat_kv]))
    q_tok = rows[:b * _G * _NQ].reshape(b, _G, _NQ, d)
    kv_tok = rows[b * _G * _NQ:].reshape(b, _G, _NKV, d)

    qi_flat = q_idx.reshape(b, _G * _NQ)
    cnt = jnp.zeros((b, n), jnp.int32).at[brows, qi_flat].add(1)
    c_slot = jnp.take_along_axis(jnp.broadcast_to(cnt[:, None, :], (b, _G, n)),
                                 q_idx, axis=2)
    inv_cnt = (1.0 / c_slot.astype(jnp.float32)).reshape(b, _G, 1, _NQ)

    nk = null_kv[0, :, :, 0, :]
    nv = null_kv[1, :, :, 0, :]
    gx = gamma_x[:, None, :, 0]
    gc = gamma_ctx[:, None, :, 0]

    ao = _grouped_attention(q_tok, kv_tok, gx, gc, nk, nv, Wq, Wkv, Wo,
                            inv_cnt)

    slots = jnp.broadcast_to(
        jnp.arange(_NQ, dtype=jnp.int32)[None, None, :], q_idx.shape)
    pos = jnp.full((b, _G, n), -1, jnp.int32)
    pos = pos.at[jnp.arange(b)[:, None, None],
                 jnp.arange(_G)[None, :, None], q_idx].set(slots)
    pos0, pos1 = pos[:, 0], pos[:, 1]
    hit0, hit1 = pos0 >= 0, pos1 >= 0
    arena = 256
    zero_base = b * _G * _NQ
    null_base = zero_base + arena
    spread = (jnp.arange(n, dtype=jnp.int32) % arena)[None, :]
    base = brows * (_G * _NQ)
    idx0 = jnp.where(hit0, base + pos0,
                     jnp.where(hit1, zero_base + spread,
                               null_base + spread)).astype(jnp.int32)
    idx1 = jnp.where(hit1, base + _NQ + pos1,
                     zero_base + spread).astype(jnp.int32)

    tab = jnp.concatenate([
        ao.reshape(b * _G * _NQ, d),
        jnp.zeros((arena, d), jnp.float32),
        jnp.broadcast_to(null_routed_token.reshape(1, d),
                         (arena, d)).astype(jnp.float32),
    ], axis=0)
    out = _sc_recombine(tab, idx0.reshape(-1), idx1.reshape(-1))
    return out.reshape(b, n, d)


# speedup vs baseline: 1.2822x; 1.0000x over previous
import functools

import jax
import jax.numpy as jnp
from jax import lax
from jax.experimental import pallas as pl
from jax.experimental.pallas import tpu as pltpu
from jax.experimental.pallas import tpu_sc as plsc

_DIM = 1024
_HEADS = 8
_DIM_HEAD = 64
_G = 2
_NQ = 1024
_NKV = 1024
_N_ITERS = 20
_FETCH_K_RATIO = 9.0 / 8.0
_EPS = 0.03
_EPS_INIT = 4.0
_EPS_DECAY = 0.7
_DI = _HEADS * _DIM_HEAD


_NW = 32
_CHUNK = 64


def _sc_gather_rows(table, idx):
    nrows = idx.shape[0]
    b_per_w = nrows // _NW
    nch = b_per_w // _CHUNK
    mesh = plsc.VectorSubcoreMesh(core_axis_name="c", subcore_axis_name="s")

    @functools.partial(
        pl.kernel, mesh=mesh,
        out_type=jax.ShapeDtypeStruct((nrows, _DIM), jnp.float32),
        scratch_types=[
            pltpu.VMEM((b_per_w,), jnp.int32),
            pltpu.VMEM((_CHUNK, _DIM), jnp.float32),
            pltpu.SemaphoreType.DMA,
        ],
    )
    def k(table_hbm, idx_hbm, out_hbm, idx_v, rows_v, gsem):
        wid = lax.axis_index("s") * 2 + lax.axis_index("c")
        base = wid * b_per_w
        pltpu.sync_copy(idx_hbm.at[pl.ds(base, b_per_w)], idx_v)
        for c in range(nch):
            pltpu.async_copy(
                table_hbm.at[idx_v.at[pl.ds(c * _CHUNK, _CHUNK)]],
                rows_v, gsem).wait()
            pltpu.sync_copy(rows_v,
                            out_hbm.at[pl.ds(base + c * _CHUNK, _CHUNK)])

    return k(table, idx)


_RCHUNK = 16


def _sc_recombine(tab, idx0, idx1):
    nrows = idx0.shape[0]
    b_per_w = nrows // _NW
    nch = b_per_w // _RCHUNK
    mesh = plsc.VectorSubcoreMesh(core_axis_name="c", subcore_axis_name="s")

    @functools.partial(
        pl.kernel, mesh=mesh,
        out_type=jax.ShapeDtypeStruct((nrows, _DIM), jnp.float32),
        scratch_types=[
            pltpu.VMEM((b_per_w,), jnp.int32),
            pltpu.VMEM((b_per_w,), jnp.int32),
            pltpu.VMEM((_RCHUNK, _DIM), jnp.float32),
            pltpu.VMEM((_RCHUNK, _DIM), jnp.float32),
            pltpu.VMEM((_RCHUNK, _DIM), jnp.float32),
            pltpu.VMEM((_RCHUNK, _DIM), jnp.float32),
            pltpu.SemaphoreType.DMA,
            pltpu.SemaphoreType.DMA,
            pltpu.SemaphoreType.DMA,
            pltpu.SemaphoreType.DMA,
            pltpu.SemaphoreType.DMA,
            pltpu.SemaphoreType.DMA,
        ],
    )
    def k(tab_hbm, i0_hbm, i1_hbm, out_hbm, i0_v, i1_v,
          r0a, r0b, r1a, r1b, sem0a, sem0b, sem1a, sem1b, wsa, wsb):
        wid = lax.axis_index("s") * 2 + lax.axis_index("c")
        base = wid * b_per_w
        pltpu.sync_copy(i0_hbm.at[pl.ds(base, b_per_w)], i0_v)
        pltpu.sync_copy(i1_hbm.at[pl.ds(base, b_per_w)], i1_v)
        r0 = (r0a, r0b)
        r1 = (r1a, r1b)
        s0 = (sem0a, sem0b)
        s1 = (sem1a, sem1b)
        ws = (wsa, wsb)

        def issue(c):
            s = c % 2
            cp0 = pltpu.async_copy(
                tab_hbm.at[i0_v.at[pl.ds(c * _RCHUNK, _RCHUNK)]],
                r0[s], s0[s])
            cp1 = pltpu.async_copy(
                tab_hbm.at[i1_v.at[pl.ds(c * _RCHUNK, _RCHUNK)]],
                r1[s], s1[s])
            return cp0, cp1

        cps = issue(0)
        wcps = [None, None]
        for c in range(nch):
            s = c % 2
            if c + 1 < nch:
                if wcps[1 - s] is not None:
                    wcps[1 - s].wait()
                    wcps[1 - s] = None
                nxt = issue(c + 1)
            else:
                nxt = None
            cps[0].wait()
            cps[1].wait()

            def row_body(row, _, _s=s):
                for j in range(_DIM // 16):
                    sl = pl.ds(j * 16, 16)
                    r0[_s][row, sl] = r0[_s][row, sl] + r1[_s][row, sl]
                return 0

            lax.fori_loop(0, _RCHUNK, row_body, 0)
            wcps[s] = pltpu.async_copy(
                r0[s], out_hbm.at[pl.ds(base + c * _RCHUNK, _RCHUNK)], ws[s])
            cps = nxt
        for w in wcps:
            if w is not None:
                w.wait()

    return k(tab, idx0, idx1)


def _coor_descent(s, k):
    logk = jnp.log(jnp.asarray(float(k), dtype=s.dtype))
    a = jnp.zeros_like(s[..., :1])
    bneg = -s
    current_eps = max(_EPS_INIT, _EPS)
    for _ in range(_N_ITERS):
        sb = (s + bneg) / current_eps
        a = current_eps * (logk - jax.nn.logsumexp(sb, axis=-1, keepdims=True))
        bneg = -jax.nn.relu(s + a)
        current_eps = max(current_eps * _EPS_DECAY, _EPS)
    return jnp.exp((s + a + bneg) / current_eps)


def _route_idx(x, routing_token, num_tokens):
    n = x.shape[1]
    s = jnp.einsum('bnd,gd->bgn', x, routing_token)
    k = min(int(num_tokens * _FETCH_K_RATIO), n)
    scores = _coor_descent(s, k)
    _, sel_idx = jax.lax.top_k(scores, num_tokens)
    return sel_idx


def _dotT(a, b):
    return lax.dot_general(a, b, (((1,), (1,)), ((), ())),
                           precision=lax.Precision.DEFAULT,
                           preferred_element_type=jnp.float32)


def _rms(t, gamma):
    ss = jnp.sum(t * t, axis=1, keepdims=True)
    normed = t / jnp.maximum(jnp.sqrt(ss), 1e-12)
    return normed * (float(_DIM) ** 0.5) * gamma


def _proj_body(qtok_ref, kvtok_ref, gx_ref, gc_ref, wq_ref, wkv_ref,
               q_ref, k_ref, v_ref):
    qn = _rms(qtok_ref[0, 0], gx_ref[...].reshape(1, _DIM))
    q_ref[0, 0] = _dotT(qn, wq_ref[0]).astype(jnp.bfloat16)
    tn = _rms(kvtok_ref[0, 0], gc_ref[...].reshape(1, _DIM))
    k_ref[0, 0] = _dotT(tn, wkv_ref[0, :_DI]).astype(jnp.bfloat16)
    v_ref[0, 0] = _dotT(tn, wkv_ref[0, _DI:]).astype(jnp.bfloat16)


_QB = 512


def _attn_body(q_ref, k_ref, v_ref, nk_ref, nv_ref, wo_ref, inv_ref,
               out_ref, o_scratch):
    qs = q_ref[0, 0] * jnp.bfloat16(_DIM_HEAD ** -0.5)
    kk = k_ref[0, 0]
    vv = v_ref[0, 0]
    nk = nk_ref[0]
    nv = nv_ref[0]
    for h in range(_HEADS):
        sl = slice(h * _DIM_HEAD, (h + 1) * _DIM_HEAD)
        qh = qs[:, sl]
        sim = _dotT(qh, kk[:, sl])
        simn = jnp.sum(qh.astype(jnp.float32) * nk[h][None, :],
                       axis=1, keepdims=True)
        m = jnp.maximum(jnp.max(sim, axis=1, keepdims=True), simn)
        p = jnp.exp(sim - m)
        pn = jnp.exp(simn - m)
        denom = jnp.sum(p, axis=1, keepdims=True) + pn
        ov = lax.dot_general(p.astype(jnp.bfloat16), vv[:, sl],
                             (((1,), (0,)), ((), ())),
                             precision=lax.Precision.DEFAULT,
                             preferred_element_type=jnp.float32)
        o_scratch[:, sl] = ((ov + pn * nv[h:h + 1]) / denom
                            ).astype(jnp.bfloat16)
    out = _dotT(o_scratch[...], wo_ref[0])
    out_ref[0, 0] = out * inv_ref[...].reshape(_QB, 1)


def _grouped_attention(q_tok, kv_tok, gamma_x, gamma_ctx, nk, nv,
                       Wq, Wkv, Wo, inv_cnt):
    b = q_tok.shape[0]
    q, kk, vv = pl.pallas_call(
        _proj_body,
        grid=(b, _G),
        in_specs=[
            pl.BlockSpec((1, 1, _NQ, _DIM), lambda i, j: (i, j, 0, 0)),
            pl.BlockSpec((1, 1, _NKV, _DIM), lambda i, j: (i, j, 0, 0)),
            pl.BlockSpec((1, 1, _DIM), lambda i, j: (j, 0, 0)),
            pl.BlockSpec((1, 1, _DIM), lambda i, j: (j, 0, 0)),
            pl.BlockSpec((1, _DI, _DIM), lambda i, j: (j, 0, 0)),
            pl.BlockSpec((1, 2 * _DI, _DIM), lambda i, j: (j, 0, 0)),
        ],
        out_specs=[
            pl.BlockSpec((1, 1, _NQ, _DI), lambda i, j: (i, j, 0, 0)),
            pl.BlockSpec((1, 1, _NKV, _DI), lambda i, j: (i, j, 0, 0)),
            pl.BlockSpec((1, 1, _NKV, _DI), lambda i, j: (i, j, 0, 0)),
        ],
        out_shape=[
            jax.ShapeDtypeStruct((b, _G, _NQ, _DI), jnp.bfloat16),
            jax.ShapeDtypeStruct((b, _G, _NKV, _DI), jnp.bfloat16),
            jax.ShapeDtypeStruct((b, _G, _NKV, _DI), jnp.bfloat16),
        ],
    )(q_tok, kv_tok, gamma_x, gamma_ctx, Wq, Wkv)
    nqb = _NQ // _QB
    return pl.pallas_call(
        _attn_body,
        grid=(b, _G, nqb),
        in_specs=[
            pl.BlockSpec((1, 1, _QB, _DI), lambda i, j, q_: (i, j, q_, 0)),
            pl.BlockSpec((1, 1, _NKV, _DI), lambda i, j, q_: (i, j, 0, 0)),
            pl.BlockSpec((1, 1, _NKV, _DI), lambda i, j, q_: (i, j, 0, 0)),
            pl.BlockSpec((1, _HEADS, _DIM_HEAD), lambda i, j, q_: (j, 0, 0)),
            pl.BlockSpec((1, _HEADS, _DIM_HEAD), lambda i, j, q_: (j, 0, 0)),
            pl.BlockSpec((1, _DIM, _DI), lambda i, j, q_: (j, 0, 0)),
            pl.BlockSpec((1, 1, 1, _QB), lambda i, j, q_: (i, j, 0, q_)),
        ],
        out_specs=pl.BlockSpec((1, 1, _QB, _DIM),
                               lambda i, j, q_: (i, j, q_, 0)),
        out_shape=jax.ShapeDtypeStruct((b, _G, _NQ, _DIM), jnp.float32),
        scratch_shapes=[pltpu.VMEM((_QB, _DI), jnp.bfloat16)],
    )(q, kk, vv, nk, nv, Wo.astype(jnp.bfloat16), inv_cnt)


def kernel(x, rt_q, rt_kv, gamma_x, gamma_ctx, null_kv, Wq, Wkv, Wo,
           null_routed_token):
    b, n, d = x.shape
    q_idx = _route_idx(x, rt_q, _NQ)
    kv_idx = _route_idx(x, rt_kv, _NKV)

    brows = jnp.arange(b)[:, None]

    flat_q = (jnp.arange(b, dtype=jnp.int32)[:, None, None] * n
              + q_idx).reshape(-1)
    flat_kv = (jnp.arange(b, dtype=jnp.int32)[:, None, None] * n
               + kv_idx).reshape(-1)
    rows = _sc_gather_rows(x.reshape(b * n, d),
                           jnp.concatenate([flat_q, flat_kv]))
    q_tok = rows[:b * _G * _NQ].reshape(b, _G, _NQ, d)
    kv_tok = rows[b * _G * _NQ:].reshape(b, _G, _NKV, d)

    qi_flat = q_idx.reshape(b, _G * _NQ)
    cnt = jnp.zeros((b, n), jnp.int32).at[brows, qi_flat].add(1)
    c_slot = jnp.take_along_axis(jnp.broadcast_to(cnt[:, None, :], (b, _G, n)),
                                 q_idx, axis=2)
    inv_cnt = (1.0 / c_slot.astype(jnp.float32)).reshape(b, _G, 1, _NQ)

    nk = null_kv[0, :, :, 0, :]
    nv = null_kv[1, :, :, 0, :]
    gx = gamma_x[:, None, :, 0]
    gc = gamma_ctx[:, None, :, 0]

    ao = _grouped_attention(q_tok, kv_tok, gx, gc, nk, nv, Wq, Wkv, Wo,
                            inv_cnt)

    slots = jnp.broadcast_to(
        jnp.arange(_NQ, dtype=jnp.int32)[None, None, :], q_idx.shape)
    pos = jnp.full((b, _G, n), -1, jnp.int32)
    pos = pos.at[jnp.arange(b)[:, None, None],
                 jnp.arange(_G)[None, :, None], q_idx].set(slots)
    pos0, pos1 = pos[:, 0], pos[:, 1]
    hit0, hit1 = pos0 >= 0, pos1 >= 0
    arena = 256
    zero_base = b * _G * _NQ
    null_base = zero_base + arena
    spread = (jnp.arange(n, dtype=jnp.int32) % arena)[None, :]
    base = brows * (_G * _NQ)
    idx0 = jnp.where(hit0, base + pos0,
                     jnp.where(hit1, zero_base + spread,
                               null_base + spread)).astype(jnp.int32)
    idx1 = jnp.where(hit1, base + _NQ + pos1,
                     zero_base + spread).astype(jnp.int32)

    tab = jnp.concatenate([
        ao.reshape(b * _G * _NQ, d),
        jnp.zeros((arena, d), jnp.float32),
        jnp.broadcast_to(null_routed_token.reshape(1, d),
                         (arena, d)).astype(jnp.float32),
    ], axis=0)
    out = _sc_recombine(tab, idx0.reshape(-1), idx1.reshape(-1))
    return out.reshape(b, n, d)
